# baseline (speedup 1.0000x reference)
import functools

import jax
import jax.numpy as jnp
from jax import lax
from jax.experimental import pallas as pl
from jax.experimental.pallas import tpu as pltpu
from jax.experimental.pallas import tpu_sc as plsc

_N = 100000
_NPAD = 100352
_E = 3200000
_CH = 80
_STRIPE = _NPAD // 16
_TBLK = 1024
_NT = 98

_mesh = plsc.VectorSubcoreMesh(core_axis_name="c", subcore_axis_name="s")


def _make_edgesum(F, feature_split, blk):
    nch = blk // _CH
    edges_per_tile = _E // 16 if feature_split else _E // 32
    blocks = edges_per_tile // blk
    assert edges_per_tile % blk == 0 and blocks >= 5

    @functools.partial(
        pl.kernel,
        out_type=jax.ShapeDtypeStruct((2, _NPAD, F), jnp.float32),
        mesh=_mesh,
        compiler_params=pltpu.CompilerParams(use_tc_tiling_on_sc=False),
        scratch_types=(
            [pltpu.VMEM((blk,), jnp.int32) for _ in range(2)]
            + [pltpu.VMEM((nch, _CH), jnp.int32) for _ in range(2)]
            + [pltpu.VMEM((blk, F), jnp.float32) for _ in range(2)]
            + [pltpu.SemaphoreType.DMA for _ in range(4)]
            + [pltpu.VMEM_SHARED((_NPAD, F), jnp.float32)]
        ),
    )
    def k(table, src, dst2d, zeros, out,
          src0, src1, dst0, dst1, rows0, rows1,
          gsem0, gsem1, ssem0, ssem1, acc):
        cid = lax.axis_index("c")
        sid = lax.axis_index("s")
        r0 = sid * _STRIPE
        pltpu.sync_copy(zeros.at[pl.ds(r0, _STRIPE), :], acc.at[pl.ds(r0, _STRIPE), :])
        plsc.subcore_barrier()

        if feature_split:
            ebase = sid * (blocks * blk)
            rbase = sid * (blocks * nch)
        else:
            tid = cid * 16 + sid
            ebase = tid * (blocks * blk)
            rbase = tid * (blocks * nch)

        bufs_a = (src0, dst0, rows0, gsem0, ssem0)
        bufs_b = (src1, dst1, rows1, gsem1, ssem1)

        def load_idx(i, b):
            src_v, dst_v = b[0], b[1]
            pltpu.sync_copy(src.at[pl.ds(ebase + i * blk, blk)], src_v)
            pltpu.sync_copy(dst2d.at[pl.ds(rbase + i * nch, nch), :], dst_v)
            if feature_split:
                offv = jnp.full((16,), cid * _NPAD, jnp.int32)
                for j2 in range(blk // 16):
                    src_v[pl.ds(j2 * 16, 16)] = src_v[pl.ds(j2 * 16, 16)] + offv

        def gather_chunk(b, j):
            return (table.at[b[0].at[pl.ds(j * _CH, _CH)]],
                    b[2].at[pl.ds(j * _CH, _CH), :], b[3])

        def scatter_chunk(b, j):
            return (b[2].at[pl.ds(j * _CH, _CH), :], acc.at[b[1].at[j]], b[4])

        def fire_gathers(b):
            for j in range(nch):
                s, d, sem = gather_chunk(b, j)
                pltpu.async_copy(s, d, sem)

        def drain(mk, b):
            for j in range(nch):
                s, d, sem = mk(b, j)
                pltpu.make_async_copy(s, d, sem).wait()

        def process(i, cur, nxt, prefetch):
            if prefetch:
                load_idx(i + 1, nxt)
                fire_gathers(nxt)
            drain(gather_chunk, cur)
            for j in range(nch):
                s, d, sem = scatter_chunk(cur, j)
                pltpu.async_copy(s, d, sem, add=True)
            drain(scatter_chunk, cur)

        load_idx(0, bufs_a)
        fire_gathers(bufs_a)

        def pair(kk, carry):
            process(2 * kk, bufs_a, bufs_b, True)
            process(2 * kk + 1, bufs_b, bufs_a, True)
            return carry

        lax.fori_loop(0, (blocks - 2) // 2, pair, 0)
        process(blocks - 2, bufs_a, bufs_b, True)
        process(blocks - 1, bufs_b, bufs_a, False)

        plsc.subcore_barrier()
        pltpu.sync_copy(acc.at[pl.ds(r0, _STRIPE), :],
                        out.at[cid, pl.ds(r0, _STRIPE), :])

    return k


_edgesum8 = _make_edgesum(8, False, 400)
_edgesum16 = _make_edgesum(16, True, 400)

_DEG_BLK = 400
_DEG_NCH = _DEG_BLK // _CH
_DEG_BLOCKS = (_E // 32) // _DEG_BLK


@functools.partial(
    pl.kernel,
    out_type=jax.ShapeDtypeStruct((2, _NPAD, 4), jnp.float32),
    mesh=_mesh,
    compiler_params=pltpu.CompilerParams(use_tc_tiling_on_sc=False),
    scratch_types=[
        pltpu.VMEM((_DEG_NCH, _CH), jnp.int32),
        pltpu.VMEM((_DEG_NCH, _CH), jnp.int32),
        pltpu.VMEM((_CH, 4), jnp.float32),
        pltpu.VMEM_SHARED((_NPAD, 4), jnp.float32),
        pltpu.SemaphoreType.DMA,
        pltpu.SemaphoreType.DMA,
    ],
)
def _degree(dst2d, ones, zeros, out, dst_a, dst_b, ones_v, acc, ssem_a, ssem_b):
    cid = lax.axis_index("c")
    sid = lax.axis_index("s")
    r0 = sid * _STRIPE
    pltpu.sync_copy(ones, ones_v)
    pltpu.sync_copy(zeros.at[pl.ds(r0, _STRIPE), :], acc.at[pl.ds(r0, _STRIPE), :])
    plsc.subcore_barrier()

    tid = cid * 16 + sid
    rbase = tid * (_DEG_BLOCKS * _DEG_NCH)

    def load_idx(i, dst_v):
        pltpu.sync_copy(dst2d.at[pl.ds(rbase + i * _DEG_NCH, _DEG_NCH), :], dst_v)

    def process(i, dst_v, ssem, dst_n, prefetch):
        for j in range(_DEG_NCH):
            pltpu.async_copy(ones_v, acc.at[dst_v.at[j]], ssem, add=True)
        if prefetch:
            load_idx(i + 1, dst_n)
        for j in range(_DEG_NCH):
            pltpu.make_async_copy(ones_v, acc.at[dst_v.at[j]], ssem).wait()

    load_idx(0, dst_a)

    def pair(kk, carry):
        process(2 * kk, dst_a, ssem_a, dst_b, True)
        process(2 * kk + 1, dst_b, ssem_b, dst_a, True)
        return carry

    lax.fori_loop(0, (_DEG_BLOCKS - 2) // 2, pair, 0)
    process(_DEG_BLOCKS - 2, dst_a, ssem_a, dst_b, True)
    process(_DEG_BLOCKS - 1, dst_b, ssem_b, dst_a, False)

    plsc.subcore_barrier()
    pltpu.sync_copy(acc.at[pl.ds(r0, _STRIPE), :],
                    out.at[cid, pl.ds(r0, _STRIPE), :])


def _t0_body(cdeg_ref, x_ref, dis_ref, xs0_ref):
    counts = cdeg_ref[0, :, 0:1] + cdeg_ref[1, :, 0:1]
    dis = lax.rsqrt(counts + 1.0)
    dis_ref[...] = dis
    xs0_ref[...] = x_ref[...] * dis


def _t1_body(a_ref, xs0_ref, dis_ref, w_ref, b_ref, out_ref):
    dis = dis_ref[...]
    agg = dis * (a_ref[0] + a_ref[1] + xs0_ref[...])
    h = jnp.maximum(
        jnp.dot(agg, w_ref[...], preferred_element_type=jnp.float32) + b_ref[...], 0.0)
    xs = dis * h
    out_ref[0] = xs[:, :16]
    out_ref[1] = xs[:, 16:]


def _t2_body(a_ref, xs1_ref, dis_ref, w2_ref, w3_ref, b2_ref, out_ref):
    dis = dis_ref[...]
    agg = jnp.concatenate(
        [dis * (a_ref[0] + xs1_ref[0]), dis * (a_ref[1] + xs1_ref[1])], axis=1)
    h2 = jnp.maximum(
        jnp.dot(agg, w2_ref[...], preferred_element_type=jnp.float32) + b2_ref[...], 0.0)
    t2 = jnp.dot(h2, w3_ref[...], preferred_element_type=jnp.float32)
    xs = dis * t2
    out_ref[0] = xs[:, :16]
    out_ref[1] = xs[:, 16:]


def _t3_body(a_ref, xs2_ref, dis_ref, b3_ref, w4_ref, out_ref):
    dis = dis_ref[...]
    agg = jnp.concatenate(
        [dis * (a_ref[0] + xs2_ref[0]), dis * (a_ref[1] + xs2_ref[1])], axis=1)
    h3 = jnp.maximum(agg + b3_ref[...], 0.0)
    out_ref[...] = dis * jnp.dot(h3, w4_ref[...], preferred_element_type=jnp.float32)


def _t4_body(a_ref, xs3_ref, dis_ref, b4_ref, out_ref):
    agg = dis_ref[...] * (a_ref[0] + a_ref[1] + xs3_ref[...])
    out_ref[...] = jnp.maximum(agg + b4_ref[...], 0.0)


def _pair_spec(F):
    return pl.BlockSpec((2, _TBLK, F), lambda i: (0, i, 0))


def _row_spec(F):
    return pl.BlockSpec((_TBLK, F), lambda i: (i, 0))


def _full_spec(shape):
    nd = len(shape)
    return pl.BlockSpec(shape, lambda i: (0,) * nd)


def kernel(x, edge_index, W1, b1, W2, b2, W3, b3, W4, b4):
    f32 = jnp.float32
    src = edge_index[0]
    dst = edge_index[1]
    dst2d = dst.reshape(_E // _CH, _CH)

    x_pad = jnp.pad(x, ((0, _NPAD - _N), (0, 5)))
    W1p = jnp.pad(W1, ((0, 5), (0, 0)))
    W4p = jnp.pad(W4, ((0, 0), (0, 5)))
    b1r = b1.reshape(1, 32)
    b2r = b2.reshape(1, 64)
    b3r = b3.reshape(1, 32)
    b4r = jnp.pad(b4, (0, 5)).reshape(1, 8)

    zeros4 = jnp.zeros((_NPAD, 4), f32)
    zeros8 = jnp.zeros((_NPAD, 8), f32)
    zeros16 = jnp.zeros((_NPAD, 16), f32)
    ones4 = jnp.ones((_CH, 4), f32)

    cdeg = _degree(dst2d, ones4, zeros4)

    dis, xs0 = pl.pallas_call(
        _t0_body,
        grid=(_NT,),
        in_specs=[_pair_spec(4), _row_spec(8)],
        out_specs=[_row_spec(1), _row_spec(8)],
        out_shape=[jax.ShapeDtypeStruct((_NPAD, 1), f32),
                   jax.ShapeDtypeStruct((_NPAD, 8), f32)],
    )(cdeg, x_pad)

    a1 = _edgesum8(xs0, src, dst2d, zeros8)

    xs1 = pl.pallas_call(
        _t1_body,
        grid=(_NT,),
        in_specs=[_pair_spec(8), _row_spec(8), _row_spec(1),
                  _full_spec((8, 32)), _full_spec((1, 32))],
        out_specs=_pair_spec(16),
        out_shape=jax.ShapeDtypeStruct((2, _NPAD, 16), f32),
    )(a1, xs0, dis, W1p, b1r)

    a2 = _edgesum16(xs1.reshape(2 * _NPAD, 16), src, dst2d, zeros16)

    xs2 = pl.pallas_call(
        _t2_body,
        grid=(_NT,),
        in_specs=[_pair_spec(16), _pair_spec(16), _row_spec(1),
                  _full_spec((32, 64)), _full_spec((64, 32)), _full_spec((1, 64))],
        out_specs=_pair_spec(16),
        out_shape=jax.ShapeDtypeStruct((2, _NPAD, 16), f32),
    )(a2, xs1, dis, W2, W3, b2r)

    a3 = _edgesum16(xs2.reshape(2 * _NPAD, 16), src, dst2d, zeros16)

    xs3 = pl.pallas_call(
        _t3_body,
        grid=(_NT,),
        in_specs=[_pair_spec(16), _pair_spec(16), _row_spec(1),
                  _full_spec((1, 32)), _full_spec((32, 8))],
        out_specs=_row_spec(8),
        out_shape=jax.ShapeDtypeStruct((_NPAD, 8), f32),
    )(a3, xs2, dis, b3r, W4p)

    a4 = _edgesum8(xs3, src, dst2d, zeros8)

    o = pl.pallas_call(
        _t4_body,
        grid=(_NT,),
        in_specs=[_pair_spec(8), _row_spec(8), _row_spec(1), _full_spec((1, 8))],
        out_specs=_row_spec(8),
        out_shape=jax.ShapeDtypeStruct((_NPAD, 8), f32),
    )(a4, xs3, dis, b4r)

    return o[:_N, :3]


# speedup vs baseline: 1.0255x; 1.0255x over previous
import functools

import jax
import jax.numpy as jnp
from jax import lax
from jax.experimental import pallas as pl
from jax.experimental.pallas import tpu as pltpu
from jax.experimental.pallas import tpu_sc as plsc

_N = 100000
_E = 3200000
_NPAD = 100352
_EPAD = 3211264
_BLK = 1024
_NCH = 8
_STRIPE = _NPAD // 16
_TBLK = 1024

_mesh = plsc.VectorSubcoreMesh(core_axis_name="c", subcore_axis_name="s")


def _make_edgesum(F, feature_split, blk):
    nch = blk // 128
    edges_per_tile = _EPAD // 16 if feature_split else _EPAD // 32
    blocks = edges_per_tile // blk
    assert edges_per_tile % blk == 0 and blocks >= 8

    @functools.partial(
        pl.kernel,
        out_type=jax.ShapeDtypeStruct((2, _NPAD, F), jnp.float32),
        mesh=_mesh,
        compiler_params=pltpu.CompilerParams(use_tc_tiling_on_sc=False),
        scratch_types=(
            [pltpu.VMEM((blk,), jnp.int32) for _ in range(3)]
            + [pltpu.VMEM((nch, 128), jnp.int32) for _ in range(3)]
            + [pltpu.VMEM((blk, F), jnp.float32) for _ in range(3)]
            + [pltpu.SemaphoreType.DMA for _ in range(9)]
            + [pltpu.VMEM_SHARED((_NPAD, F), jnp.float32)]
        ),
    )
    def k(table, src, dst2d, zeros, out,
          src0, src1, src2, dst0, dst1, dst2, rows0, rows1, rows2,
          gsem0, gsem1, gsem2, ssem0, ssem1, ssem2, isem0, isem1, isem2, acc):
        cid = lax.axis_index("c")
        sid = lax.axis_index("s")
        r0 = sid * _STRIPE
        pltpu.sync_copy(zeros.at[pl.ds(r0, _STRIPE), :], acc.at[pl.ds(r0, _STRIPE), :])
        plsc.subcore_barrier()

        if feature_split:
            ebase = sid * (blocks * blk)
            rbase = sid * (blocks * nch)
        else:
            tid = cid * 16 + sid
            ebase = tid * (blocks * blk)
            rbase = tid * (blocks * nch)

        bufs = [
            (src0, dst0, rows0, gsem0, ssem0, isem0),
            (src1, dst1, rows1, gsem1, ssem1, isem1),
            (src2, dst2, rows2, gsem2, ssem2, isem2),
        ]

        def idx_copies(i, b):
            return [
                (src.at[pl.ds(ebase + i * blk, blk)], b[0], b[5]),
                (dst2d.at[pl.ds(rbase + i * nch, nch), :], b[1], b[5]),
            ]

        def fire_idx(i, b):
            for s_, d_, sem in idx_copies(i, b):
                pltpu.async_copy(s_, d_, sem)

        def wait_idx(i, b):
            for s_, d_, sem in idx_copies(i, b):
                pltpu.make_async_copy(s_, d_, sem).wait()
            if feature_split:
                src_v = b[0]
                offv = jnp.full((16,), cid * _NPAD, jnp.int32)
                for j2 in range(blk // 16):
                    src_v[pl.ds(j2 * 16, 16)] = src_v[pl.ds(j2 * 16, 16)] + offv

        def gather_chunk(b, j):
            return (table.at[b[0].at[pl.ds(j * 128, 128)]],
                    b[2].at[pl.ds(j * 128, 128), :], b[3])

        def scatter_chunk(b, j):
            return (b[2].at[pl.ds(j * 128, 128), :], acc.at[b[1].at[j]], b[4])

        def fire_gathers(b):
            for j in range(nch):
                s_, d_, sem = gather_chunk(b, j)
                pltpu.async_copy(s_, d_, sem)

        def drain(mk, b):
            for j in range(nch):
                s_, d_, sem = mk(b, j)
                pltpu.make_async_copy(s_, d_, sem).wait()

        def process(i, kcur, do_b, do_cd, do_f):
            cur = bufs[kcur]
            nxt = bufs[(kcur + 1) % 3]
            prv = bufs[(kcur + 2) % 3]
            drain(gather_chunk, cur)
            if do_b:
                drain(scatter_chunk, prv)
            if do_cd:
                wait_idx(i + 1, nxt)
                fire_gathers(nxt)
            for j in range(nch):
                s_, d_, sem = scatter_chunk(cur, j)
                pltpu.async_copy(s_, d_, sem, add=True)
            if do_f:
                fire_idx(i + 2, prv)

        for s_, d_, sem in idx_copies(0, bufs[0]):
            pltpu.sync_copy(s_, d_)
        if feature_split:
            offv = jnp.full((16,), cid * _NPAD, jnp.int32)
            for j2 in range(blk // 16):
                src0[pl.ds(j2 * 16, 16)] = src0[pl.ds(j2 * 16, 16)] + offv
        fire_gathers(bufs[0])
        fire_idx(1, bufs[1])
        process(0, 0, False, True, True)

        mt = max(0, (blocks - 4) // 3)

        def triple(kk, carry):
            base = 3 * kk + 1
            process(base, 1, True, True, True)
            process(base + 1, 2, True, True, True)
            process(base + 2, 0, True, True, True)
            return carry

        lax.fori_loop(0, mt, triple, 0)
        for i in range(3 * mt + 1, blocks):
            process(i, i % 3, True, i < blocks - 1, i + 2 <= blocks - 1)
        drain(scatter_chunk, bufs[(blocks - 1) % 3])

        plsc.subcore_barrier()
        pltpu.sync_copy(acc.at[pl.ds(r0, _STRIPE), :],
                        out.at[cid, pl.ds(r0, _STRIPE), :])

    return k


_edgesum8 = _make_edgesum(8, False, 1024)
_edgesum16 = _make_edgesum(16, True, 256)


@functools.partial(
    pl.kernel,
    out_type=jax.ShapeDtypeStruct((2, _NPAD, 4), jnp.float32),
    mesh=_mesh,
    compiler_params=pltpu.CompilerParams(use_tc_tiling_on_sc=False),
    scratch_types=[
        pltpu.VMEM((_NCH, 128), jnp.int32),
        pltpu.VMEM((_NCH, 128), jnp.int32),
        pltpu.VMEM((128, 4), jnp.float32),
        pltpu.VMEM_SHARED((_NPAD, 4), jnp.float32),
        pltpu.SemaphoreType.DMA,
        pltpu.SemaphoreType.DMA,
    ],
)
def _degree(dst2d, ones, zeros, out, dst_a, dst_b, ones_v, acc, ssem_a, ssem_b):
    cid = lax.axis_index("c")
    sid = lax.axis_index("s")
    r0 = sid * _STRIPE
    pltpu.sync_copy(ones, ones_v)
    pltpu.sync_copy(zeros.at[pl.ds(r0, _STRIPE), :], acc.at[pl.ds(r0, _STRIPE), :])
    plsc.subcore_barrier()

    tid = cid * 16 + sid
    rbase = tid * (98 * _NCH)

    def load_idx(i, dst_v):
        pltpu.sync_copy(dst2d.at[pl.ds(rbase + i * _NCH, _NCH), :], dst_v)

    def process(i, dst_v, ssem, dst_n, prefetch):
        for j in range(_NCH):
            pltpu.async_copy(ones_v, acc.at[dst_v.at[j]], ssem, add=True)
        if prefetch:
            load_idx(i + 1, dst_n)
        for j in range(_NCH):
            pltpu.make_async_copy(ones_v, acc.at[dst_v.at[j]], ssem).wait()

    load_idx(0, dst_a)

    def pair(kk, carry):
        process(2 * kk, dst_a, ssem_a, dst_b, True)
        process(2 * kk + 1, dst_b, ssem_b, dst_a, True)
        return carry

    lax.fori_loop(0, 48, pair, 0)
    process(96, dst_a, ssem_a, dst_b, True)
    process(97, dst_b, ssem_b, dst_a, False)

    plsc.subcore_barrier()
    pltpu.sync_copy(acc.at[pl.ds(r0, _STRIPE), :],
                    out.at[cid, pl.ds(r0, _STRIPE), :])


def _t0_body(cdeg_ref, x_ref, dis_ref, xs0_ref):
    i = pl.program_id(0)
    counts = cdeg_ref[0, :, 0:1] + cdeg_ref[1, :, 0:1]
    row = lax.broadcasted_iota(jnp.int32, (_TBLK, 1), 0) + i * _TBLK
    dis = jnp.where(row < _N, lax.rsqrt(counts + 1.0), 0.0)
    dis_ref[...] = dis
    xs0_ref[...] = x_ref[...] * dis


def _t1_body(a_ref, xs0_ref, dis_ref, w_ref, b_ref, out_ref):
    dis = dis_ref[...]
    agg = dis * (a_ref[0] + a_ref[1] + xs0_ref[...])
    h = jnp.maximum(
        jnp.dot(agg, w_ref[...], preferred_element_type=jnp.float32) + b_ref[...], 0.0)
    xs = dis * h
    out_ref[0] = xs[:, :16]
    out_ref[1] = xs[:, 16:]


def _t2_body(a_ref, xs1_ref, dis_ref, w2_ref, w3_ref, b2_ref, out_ref):
    dis = dis_ref[...]
    agg = jnp.concatenate(
        [dis * (a_ref[0] + xs1_ref[0]), dis * (a_ref[1] + xs1_ref[1])], axis=1)
    h2 = jnp.maximum(
        jnp.dot(agg, w2_ref[...], preferred_element_type=jnp.float32) + b2_ref[...], 0.0)
    t2 = jnp.dot(h2, w3_ref[...], preferred_element_type=jnp.float32)
    xs = dis * t2
    out_ref[0] = xs[:, :16]
    out_ref[1] = xs[:, 16:]


def _t3_body(a_ref, xs2_ref, dis_ref, b3_ref, w4_ref, out_ref):
    dis = dis_ref[...]
    agg = jnp.concatenate(
        [dis * (a_ref[0] + xs2_ref[0]), dis * (a_ref[1] + xs2_ref[1])], axis=1)
    h3 = jnp.maximum(agg + b3_ref[...], 0.0)
    out_ref[...] = dis * jnp.dot(h3, w4_ref[...], preferred_element_type=jnp.float32)


def _t4_body(a_ref, xs3_ref, dis_ref, b4_ref, out_ref):
    agg = dis_ref[...] * (a_ref[0] + a_ref[1] + xs3_ref[...])
    out_ref[...] = jnp.maximum(agg + b4_ref[...], 0.0)


def _pair_spec(F):
    return pl.BlockSpec((2, _TBLK, F), lambda i: (0, i, 0))


def _row_spec(F):
    return pl.BlockSpec((_TBLK, F), lambda i: (i, 0))


def _full_spec(shape):
    nd = len(shape)
    return pl.BlockSpec(shape, lambda i: (0,) * nd)


def kernel(x, edge_index, W1, b1, W2, b2, W3, b3, W4, b4):
    f32 = jnp.float32
    src = edge_index[0]
    dst = edge_index[1]
    pad_idx = jnp.full((_EPAD - _E,), _NPAD - 1, jnp.int32)
    srcp = jnp.concatenate([src, pad_idx])
    dst2d = jnp.concatenate([dst, pad_idx]).reshape(_EPAD // 128, 128)

    x_pad = jnp.pad(x, ((0, _NPAD - _N), (0, 5)))
    W1p = jnp.pad(W1, ((0, 5), (0, 0)))
    W4p = jnp.pad(W4, ((0, 0), (0, 5)))
    b1r = b1.reshape(1, 32)
    b2r = b2.reshape(1, 64)
    b3r = b3.reshape(1, 32)
    b4r = jnp.pad(b4, (0, 5)).reshape(1, 8)

    zeros4 = jnp.zeros((_NPAD, 4), f32)
    zeros8 = jnp.zeros((_NPAD, 8), f32)
    zeros16 = jnp.zeros((_NPAD, 16), f32)
    ones4 = jnp.ones((128, 4), f32)

    cdeg = _degree(dst2d, ones4, zeros4)

    dis, xs0 = pl.pallas_call(
        _t0_body,
        grid=(_NPAD // _TBLK,),
        in_specs=[_pair_spec(4), _row_spec(8)],
        out_specs=[_row_spec(1), _row_spec(8)],
        out_shape=[jax.ShapeDtypeStruct((_NPAD, 1), f32),
                   jax.ShapeDtypeStruct((_NPAD, 8), f32)],
    )(cdeg, x_pad)

    a1 = _edgesum8(xs0, srcp, dst2d, zeros8)

    xs1 = pl.pallas_call(
        _t1_body,
        grid=(_NPAD // _TBLK,),
        in_specs=[_pair_spec(8), _row_spec(8), _row_spec(1),
                  _full_spec((8, 32)), _full_spec((1, 32))],
        out_specs=_pair_spec(16),
        out_shape=jax.ShapeDtypeStruct((2, _NPAD, 16), f32),
    )(a1, xs0, dis, W1p, b1r)

    a2 = _edgesum16(xs1.reshape(2 * _NPAD, 16), srcp, dst2d, zeros16)

    xs2 = pl.pallas_call(
        _t2_body,
        grid=(_NPAD // _TBLK,),
        in_specs=[_pair_spec(16), _pair_spec(16), _row_spec(1),
                  _full_spec((32, 64)), _full_spec((64, 32)), _full_spec((1, 64))],
        out_specs=_pair_spec(16),
        out_shape=jax.ShapeDtypeStruct((2, _NPAD, 16), f32),
    )(a2, xs1, dis, W2, W3, b2r)

    a3 = _edgesum16(xs2.reshape(2 * _NPAD, 16), srcp, dst2d, zeros16)

    xs3 = pl.pallas_call(
        _t3_body,
        grid=(_NPAD // _TBLK,),
        in_specs=[_pair_spec(16), _pair_spec(16), _row_spec(1),
                  _full_spec((1, 32)), _full_spec((32, 8))],
        out_specs=_row_spec(8),
        out_shape=jax.ShapeDtypeStruct((_NPAD, 8), f32),
    )(a3, xs2, dis, b3r, W4p)

    a4 = _edgesum8(xs3, srcp, dst2d, zeros8)

    o = pl.pallas_call(
        _t4_body,
        grid=(_NPAD // _TBLK,),
        in_specs=[_pair_spec(8), _row_spec(8), _row_spec(1), _full_spec((1, 8))],
        out_specs=_row_spec(8),
        out_shape=jax.ShapeDtypeStruct((_NPAD, 8), f32),
    )(a4, xs3, dis, b4r)

    return o[:_N, :3]


# speedup vs baseline: 1.1685x; 1.1394x over previous
import functools

import jax
import jax.numpy as jnp
from jax import lax
from jax.experimental import pallas as pl
from jax.experimental.pallas import tpu as pltpu
from jax.experimental.pallas import tpu_sc as plsc

_N = 100000
_E = 3200000
_NPAD = 100352
_EPAD = 3211264
_BLK = 1024
_NCH = 8
_STRIPE = _NPAD // 16
_TBLK = 1024

_mesh = plsc.VectorSubcoreMesh(core_axis_name="c", subcore_axis_name="s")


def _make_edgesum(F, feature_split, blk):
    nch = blk // 128
    edges_per_tile = _EPAD // 16 if feature_split else _EPAD // 32
    blocks = edges_per_tile // blk
    assert edges_per_tile % blk == 0 and blocks >= 8

    @functools.partial(
        pl.kernel,
        out_type=jax.ShapeDtypeStruct((2, _NPAD, F), jnp.float32),
        mesh=_mesh,
        compiler_params=pltpu.CompilerParams(use_tc_tiling_on_sc=False),
        scratch_types=(
            [pltpu.VMEM((blk,), jnp.int32) for _ in range(3)]
            + [pltpu.VMEM((nch, 128), jnp.int32) for _ in range(3)]
            + [pltpu.VMEM((blk, F), jnp.float32) for _ in range(3)]
            + [pltpu.SemaphoreType.DMA for _ in range(9)]
            + [pltpu.VMEM_SHARED((_NPAD, F), jnp.float32)]
        ),
    )
    def k(table, src, dst2d, zeros, out,
          src0, src1, src2, dst0, dst1, dst2, rows0, rows1, rows2,
          gsem0, gsem1, gsem2, ssem0, ssem1, ssem2, isem0, isem1, isem2, acc):
        cid = lax.axis_index("c")
        sid = lax.axis_index("s")
        r0 = sid * _STRIPE
        pltpu.sync_copy(zeros.at[pl.ds(r0, _STRIPE), :], acc.at[pl.ds(r0, _STRIPE), :])
        plsc.subcore_barrier()

        if feature_split:
            ebase = sid * (blocks * blk)
            rbase = sid * (blocks * nch)
        else:
            tid = cid * 16 + sid
            ebase = tid * (blocks * blk)
            rbase = tid * (blocks * nch)

        bufs = [
            (src0, dst0, rows0, gsem0, ssem0, isem0),
            (src1, dst1, rows1, gsem1, ssem1, isem1),
            (src2, dst2, rows2, gsem2, ssem2, isem2),
        ]

        def idx_copies(i, b):
            return [
                (src.at[pl.ds(ebase + i * blk, blk)], b[0], b[5]),
                (dst2d.at[pl.ds(rbase + i * nch, nch), :], b[1], b[5]),
            ]

        def fire_idx(i, b):
            for s_, d_, sem in idx_copies(i, b):
                pltpu.async_copy(s_, d_, sem)

        def wait_idx(i, b):
            for s_, d_, sem in idx_copies(i, b):
                pltpu.make_async_copy(s_, d_, sem).wait()
            if feature_split:
                src_v = b[0]
                offv = jnp.full((16,), cid * _NPAD, jnp.int32)
                for j2 in range(blk // 16):
                    src_v[pl.ds(j2 * 16, 16)] = src_v[pl.ds(j2 * 16, 16)] + offv

        def gather_chunk(b, j):
            return (table.at[b[0].at[pl.ds(j * 128, 128)]],
                    b[2].at[pl.ds(j * 128, 128), :], b[3])

        def scatter_chunk(b, j):
            return (b[2].at[pl.ds(j * 128, 128), :], acc.at[b[1].at[j]], b[4])

        def fire_gathers(b):
            for j in range(nch):
                s_, d_, sem = gather_chunk(b, j)
                pltpu.async_copy(s_, d_, sem)

        def drain(mk, b):
            for j in range(nch):
                s_, d_, sem = mk(b, j)
                pltpu.make_async_copy(s_, d_, sem).wait()

        def process(i, kcur, do_b, do_cd, do_f):
            cur = bufs[kcur]
            nxt = bufs[(kcur + 1) % 3]
            prv = bufs[(kcur + 2) % 3]
            drain(gather_chunk, cur)
            if do_b:
                drain(scatter_chunk, prv)
            if do_cd:
                wait_idx(i + 1, nxt)
                fire_gathers(nxt)
            for j in range(nch):
                s_, d_, sem = scatter_chunk(cur, j)
                pltpu.async_copy(s_, d_, sem, add=True)
            if do_f:
                fire_idx(i + 2, prv)

        for s_, d_, sem in idx_copies(0, bufs[0]):
            pltpu.sync_copy(s_, d_)
        if feature_split:
            offv = jnp.full((16,), cid * _NPAD, jnp.int32)
            for j2 in range(blk // 16):
                src0[pl.ds(j2 * 16, 16)] = src0[pl.ds(j2 * 16, 16)] + offv
        fire_gathers(bufs[0])
        fire_idx(1, bufs[1])
        process(0, 0, False, True, True)

        mt = max(0, (blocks - 4) // 3)

        def triple(kk, carry):
            base = 3 * kk + 1
            process(base, 1, True, True, True)
            process(base + 1, 2, True, True, True)
            process(base + 2, 0, True, True, True)
            return carry

        lax.fori_loop(0, mt, triple, 0)
        for i in range(3 * mt + 1, blocks):
            process(i, i % 3, True, i < blocks - 1, i + 2 <= blocks - 1)
        drain(scatter_chunk, bufs[(blocks - 1) % 3])

        plsc.subcore_barrier()
        pltpu.sync_copy(acc.at[pl.ds(r0, _STRIPE), :],
                        out.at[cid, pl.ds(r0, _STRIPE), :])

    return k


_edgesum8 = _make_edgesum(8, False, 1024)
_edgesum16 = _make_edgesum(16, True, 512)


@functools.partial(
    pl.kernel,
    out_type=jax.ShapeDtypeStruct((2, _NPAD, 4), jnp.float32),
    mesh=_mesh,
    compiler_params=pltpu.CompilerParams(use_tc_tiling_on_sc=False),
    scratch_types=[
        pltpu.VMEM((_NCH, 128), jnp.int32),
        pltpu.VMEM((_NCH, 128), jnp.int32),
        pltpu.VMEM((128, 4), jnp.float32),
        pltpu.VMEM_SHARED((_NPAD, 4), jnp.float32),
        pltpu.SemaphoreType.DMA,
        pltpu.SemaphoreType.DMA,
    ],
)
def _degree(dst2d, ones, zeros, out, dst_a, dst_b, ones_v, acc, ssem_a, ssem_b):
    cid = lax.axis_index("c")
    sid = lax.axis_index("s")
    r0 = sid * _STRIPE
    pltpu.sync_copy(ones, ones_v)
    pltpu.sync_copy(zeros.at[pl.ds(r0, _STRIPE), :], acc.at[pl.ds(r0, _STRIPE), :])
    plsc.subcore_barrier()

    tid = cid * 16 + sid
    rbase = tid * (98 * _NCH)

    def load_idx(i, dst_v):
        pltpu.sync_copy(dst2d.at[pl.ds(rbase + i * _NCH, _NCH), :], dst_v)

    def process(i, dst_v, ssem, dst_n, prefetch):
        for j in range(_NCH):
            pltpu.async_copy(ones_v, acc.at[dst_v.at[j]], ssem, add=True)
        if prefetch:
            load_idx(i + 1, dst_n)
        for j in range(_NCH):
            pltpu.make_async_copy(ones_v, acc.at[dst_v.at[j]], ssem).wait()

    load_idx(0, dst_a)

    def pair(kk, carry):
        process(2 * kk, dst_a, ssem_a, dst_b, True)
        process(2 * kk + 1, dst_b, ssem_b, dst_a, True)
        return carry

    lax.fori_loop(0, 48, pair, 0)
    process(96, dst_a, ssem_a, dst_b, True)
    process(97, dst_b, ssem_b, dst_a, False)

    plsc.subcore_barrier()
    pltpu.sync_copy(acc.at[pl.ds(r0, _STRIPE), :],
                    out.at[cid, pl.ds(r0, _STRIPE), :])


def _t0_body(cdeg_ref, x_ref, dis_ref, xs0_ref):
    i = pl.program_id(0)
    counts = cdeg_ref[0, :, 0:1] + cdeg_ref[1, :, 0:1]
    row = lax.broadcasted_iota(jnp.int32, (_TBLK, 1), 0) + i * _TBLK
    dis = jnp.where(row < _N, lax.rsqrt(counts + 1.0), 0.0)
    dis_ref[...] = dis
    xs0_ref[...] = x_ref[...] * dis


def _t1_body(a_ref, xs0_ref, dis_ref, w_ref, b_ref, out_ref):
    dis = dis_ref[...]
    agg = dis * (a_ref[0] + a_ref[1] + xs0_ref[...])
    h = jnp.maximum(
        jnp.dot(agg, w_ref[...], preferred_element_type=jnp.float32) + b_ref[...], 0.0)
    xs = dis * h
    out_ref[0] = xs[:, :16]
    out_ref[1] = xs[:, 16:]


def _t2_body(a_ref, xs1_ref, dis_ref, w2_ref, w3_ref, b2_ref, out_ref):
    dis = dis_ref[...]
    agg = jnp.concatenate(
        [dis * (a_ref[0] + xs1_ref[0]), dis * (a_ref[1] + xs1_ref[1])], axis=1)
    h2 = jnp.maximum(
        jnp.dot(agg, w2_ref[...], preferred_element_type=jnp.float32) + b2_ref[...], 0.0)
    t2 = jnp.dot(h2, w3_ref[...], preferred_element_type=jnp.float32)
    xs = dis * t2
    out_ref[0] = xs[:, :16]
    out_ref[1] = xs[:, 16:]


def _t3_body(a_ref, xs2_ref, dis_ref, b3_ref, w4_ref, out_ref):
    dis = dis_ref[...]
    agg = jnp.concatenate(
        [dis * (a_ref[0] + xs2_ref[0]), dis * (a_ref[1] + xs2_ref[1])], axis=1)
    h3 = jnp.maximum(agg + b3_ref[...], 0.0)
    out_ref[...] = dis * jnp.dot(h3, w4_ref[...], preferred_element_type=jnp.float32)


def _t4_body(a_ref, xs3_ref, dis_ref, b4_ref, out_ref):
    agg = dis_ref[...] * (a_ref[0] + a_ref[1] + xs3_ref[...])
    out_ref[...] = jnp.maximum(agg + b4_ref[...], 0.0)


def _pair_spec(F):
    return pl.BlockSpec((2, _TBLK, F), lambda i: (0, i, 0))


def _row_spec(F):
    return pl.BlockSpec((_TBLK, F), lambda i: (i, 0))


def _full_spec(shape):
    nd = len(shape)
    return pl.BlockSpec(shape, lambda i: (0,) * nd)


def kernel(x, edge_index, W1, b1, W2, b2, W3, b3, W4, b4):
    f32 = jnp.float32
    src = edge_index[0]
    dst = edge_index[1]
    pad_idx = jnp.full((_EPAD - _E,), _NPAD - 1, jnp.int32)
    srcp = jnp.concatenate([src, pad_idx])
    dst2d = jnp.concatenate([dst, pad_idx]).reshape(_EPAD // 128, 128)

    x_pad = jnp.pad(x, ((0, _NPAD - _N), (0, 5)))
    W1p = jnp.pad(W1, ((0, 5), (0, 0)))
    W4p = jnp.pad(W4, ((0, 0), (0, 5)))
    b1r = b1.reshape(1, 32)
    b2r = b2.reshape(1, 64)
    b3r = b3.reshape(1, 32)
    b4r = jnp.pad(b4, (0, 5)).reshape(1, 8)

    zeros4 = jnp.zeros((_NPAD, 4), f32)
    zeros8 = jnp.zeros((_NPAD, 8), f32)
    zeros16 = jnp.zeros((_NPAD, 16), f32)
    ones4 = jnp.ones((128, 4), f32)

    cdeg = _degree(dst2d, ones4, zeros4)

    dis, xs0 = pl.pallas_call(
        _t0_body,
        grid=(_NPAD // _TBLK,),
        in_specs=[_pair_spec(4), _row_spec(8)],
        out_specs=[_row_spec(1), _row_spec(8)],
        out_shape=[jax.ShapeDtypeStruct((_NPAD, 1), f32),
                   jax.ShapeDtypeStruct((_NPAD, 8), f32)],
    )(cdeg, x_pad)

    a1 = _edgesum8(xs0, srcp, dst2d, zeros8)

    xs1 = pl.pallas_call(
        _t1_body,
        grid=(_NPAD // _TBLK,),
        in_specs=[_pair_spec(8), _row_spec(8), _row_spec(1),
                  _full_spec((8, 32)), _full_spec((1, 32))],
        out_specs=_pair_spec(16),
        out_shape=jax.ShapeDtypeStruct((2, _NPAD, 16), f32),
    )(a1, xs0, dis, W1p, b1r)

    a2 = _edgesum16(xs1.reshape(2 * _NPAD, 16), srcp, dst2d, zeros16)

    xs2 = pl.pallas_call(
        _t2_body,
        grid=(_NPAD // _TBLK,),
        in_specs=[_pair_spec(16), _pair_spec(16), _row_spec(1),
                  _full_spec((32, 64)), _full_spec((64, 32)), _full_spec((1, 64))],
        out_specs=_pair_spec(16),
        out_shape=jax.ShapeDtypeStruct((2, _NPAD, 16), f32),
    )(a2, xs1, dis, W2, W3, b2r)

    a3 = _edgesum16(xs2.reshape(2 * _NPAD, 16), srcp, dst2d, zeros16)

    xs3 = pl.pallas_call(
        _t3_body,
        grid=(_NPAD // _TBLK,),
        in_specs=[_pair_spec(16), _pair_spec(16), _row_spec(1),
                  _full_spec((1, 32)), _full_spec((32, 8))],
        out_specs=_row_spec(8),
        out_shape=jax.ShapeDtypeStruct((_NPAD, 8), f32),
    )(a3, xs2, dis, b3r, W4p)

    a4 = _edgesum8(xs3, srcp, dst2d, zeros8)

    o = pl.pallas_call(
        _t4_body,
        grid=(_NPAD // _TBLK,),
        in_specs=[_pair_spec(8), _row_spec(8), _row_spec(1), _full_spec((1, 8))],
        out_specs=_row_spec(8),
        out_shape=jax.ShapeDtypeStruct((_NPAD, 8), f32),
    )(a4, xs3, dis, b4r)

    return o[:_N, :3]
